# baseline (speedup 1.0000x reference)
import functools

import jax
import jax.numpy as jnp
from jax import lax
from jax.experimental import pallas as pl
from jax.experimental.pallas import tpu as pltpu
from jax.experimental.pallas import tpu_sc as plsc

B, V, E, FIN, K, FOUT = 4, 10000, 160000, 256, 5, 256

NC, NS, L = 2, 16, 16

W = 128
NCH = (B * FIN) // W
CPC = NCH // NC
EB = 80
SB = 3 * EB
EPT = E // NS
NBLK = 126
EPT_P = NBLK * EB
RPT = V // NS
RB = 25
NRB = RPT // RB


def _spmm_body(x_hbm, ei3_hbm, prev_hbm, y_hbm,
               acc,
               gidx0, srow0, rows0,
               gidx1, srow1, rows1,
               gidx2, srow2, rows2,
               sbuf0, sbuf1,
               wbv0, pbv0, wbv1, pbv1,
               gsem0, ssem0, gsem1, ssem1, gsem2, ssem2,
               esem0, esem1,
               lsem0, stsem0, lsem1, stsem1,
               *, has_prev):
  cid = lax.axis_index("c")
  sid = lax.axis_index("s")
  ebase = sid * EPT_P

  E0 = (gidx0, srow0, rows0, gsem0, ssem0)
  E1 = (gidx1, srow1, rows1, gsem1, ssem1)
  E2 = (gidx2, srow2, rows2, gsem2, ssem2)

  def slab_issue(s, sbuf, esem):
    pltpu.async_copy(ei3_hbm.at[:, pl.ds(ebase + s * SB, SB)], sbuf, esem)

  def slab_wait(sbuf, esem):
    pltpu.make_async_copy(
        ei3_hbm.at[:, pl.ds(ebase, SB)], sbuf, esem).wait()

  def gather_issue(b, sbuf, boff, cbase):
    for j in range(EB // L):
      b[0][pl.ds(j * L, L)] = sbuf[0, pl.ds(boff + j * L, L)] + cbase
    pltpu.async_copy(x_hbm.at[b[0]], b[2], b[3])

  def gather_wait(b):
    pltpu.make_async_copy(x_hbm.at[b[0]], b[2], b[3]).wait()

  def scatter_issue(b, sbuf, boff):
    for j in range(EB // L):
      b[1][pl.ds(j * L, L)] = sbuf[1, pl.ds(boff + j * L, L)]
    pltpu.async_copy(b[2], acc.at[b[1]], b[4], add=True)

  def scatter_wait(b):
    pltpu.make_async_copy(b[2], acc.at[b[1]], b[4]).wait()

  def scale(b, sbuf, boff):
    @pl.loop(0, EB // L)
    def _sg(g):
      v16 = plsc.bitcast(sbuf[2, pl.ds(boff + g * L, L)], jnp.float32)
      if has_prev:
        v16 = v16 * 2.0
      for i in range(L):
        val = v16[i]
        e = g * L + i
        for j in range(W // L):
          sl = pl.ds(j * L, L)
          b[2][e, sl] = b[2][e, sl] * val

  def step3(blk, A, Bp, own, pf, cbase):
    @pl.when(blk + 2 < NBLK)
    def _pf():
      @pl.when(blk >= 1)
      def _dr():
        scatter_wait(Bp)
      gather_issue(Bp, pf[0], pf[1], cbase)
    gather_wait(A)
    scale(A, own[0], own[1])
    scatter_issue(A, own[0], own[1])

  @pl.loop(0, CPC)
  def _chunk_loop(ci):
    chunk = cid * CPC + ci
    cbase = chunk * V

    @pl.loop(0, RB)
    def _zfill(i):
      for j in range(W // L):
        wbv0[i, pl.ds(j * L, L)] = jnp.zeros((L,), jnp.float32)

    @pl.loop(0, NRB)
    def _zissue(wb):
      pltpu.sync_copy(wbv0, acc.at[pl.ds(sid * RPT + wb * RB, RB)])

    plsc.subcore_barrier()

    slab_issue(0, sbuf0, esem0)
    slab_issue(1, sbuf1, esem1)
    slab_wait(sbuf0, esem0)
    gather_issue(E0, sbuf0, 0, cbase)
    gather_issue(E1, sbuf0, EB, cbase)

    @pl.loop(0, NBLK // 6)
    def _six(j):
      blk = 6 * j
      slab_wait(sbuf1, esem1)
      step3(blk + 0, E0, E2, (sbuf0, 0), (sbuf0, 2 * EB), cbase)
      step3(blk + 1, E1, E0, (sbuf0, EB), (sbuf1, 0), cbase)
      step3(blk + 2, E2, E1, (sbuf0, 2 * EB), (sbuf1, EB), cbase)

      @pl.when(blk + 6 < NBLK)
      def _s0():
        slab_issue(2 * j + 2, sbuf0, esem0)

      step3(blk + 3, E0, E2, (sbuf1, 0), (sbuf1, 2 * EB), cbase)

      @pl.when(blk + 6 < NBLK)
      def _w0():
        slab_wait(sbuf0, esem0)

      step3(blk + 4, E1, E0, (sbuf1, EB), (sbuf0, 0), cbase)
      step3(blk + 5, E2, E1, (sbuf1, 2 * EB), (sbuf0, EB), cbase)

      @pl.when(blk + 6 < NBLK)
      def _s1():
        slab_issue(2 * j + 3, sbuf1, esem1)

    scatter_wait(E0)
    scatter_wait(E1)
    scatter_wait(E2)

    plsc.subcore_barrier()

    @pl.loop(0, NRB // 2)
    def _wb(i):
      wb = 2 * i
      r0 = sid * RPT + wb * RB
      r1 = r0 + RB
      if has_prev:
        dpa = pltpu.async_copy(prev_hbm.at[pl.ds(cbase + r0, RB)], pbv0, lsem0)
        dpb = pltpu.async_copy(prev_hbm.at[pl.ds(cbase + r1, RB)], pbv1, lsem1)
      pltpu.sync_copy(acc.at[pl.ds(r0, RB)], wbv0)
      if has_prev:
        dpa.wait()

        @pl.loop(0, RB)
        def _sub0(r):
          for j in range(W // L):
            sl = pl.ds(j * L, L)
            wbv0[r, sl] = wbv0[r, sl] - pbv0[r, sl]

      sta = pltpu.async_copy(wbv0, y_hbm.at[pl.ds(cbase + r0, RB)], stsem0)
      pltpu.sync_copy(acc.at[pl.ds(r1, RB)], wbv1)
      if has_prev:
        dpb.wait()

        @pl.loop(0, RB)
        def _sub1(r):
          for j in range(W // L):
            sl = pl.ds(j * L, L)
            wbv1[r, sl] = wbv1[r, sl] - pbv1[r, sl]

      stb = pltpu.async_copy(wbv1, y_hbm.at[pl.ds(cbase + r1, RB)], stsem1)
      sta.wait()
      stb.wait()

    wbt = NRB - 1
    rt = sid * RPT + wbt * RB
    pltpu.sync_copy(acc.at[pl.ds(rt, RB)], wbv0)
    if has_prev:
      pltpu.sync_copy(prev_hbm.at[pl.ds(cbase + rt, RB)], pbv0)

      @pl.loop(0, RB)
      def _subt(r):
        for j in range(W // L):
          sl = pl.ds(j * L, L)
          wbv0[r, sl] = wbv0[r, sl] - pbv0[r, sl]

    pltpu.sync_copy(wbv0, y_hbm.at[pl.ds(cbase + rt, RB)])

    plsc.subcore_barrier()


def _make_spmm(has_prev):
  mesh = plsc.VectorSubcoreMesh(core_axis_name="c", subcore_axis_name="s")
  return pl.kernel(
      functools.partial(_spmm_body, has_prev=has_prev),
      out_type=jax.ShapeDtypeStruct((NCH * V, W), jnp.float32),
      mesh=mesh,
      scratch_types=[
          pltpu.VMEM_SHARED((V, W), jnp.float32),
          pltpu.VMEM((EB,), jnp.int32),
          pltpu.VMEM((EB,), jnp.int32),
          pltpu.VMEM((EB, W), jnp.float32),
          pltpu.VMEM((EB,), jnp.int32),
          pltpu.VMEM((EB,), jnp.int32),
          pltpu.VMEM((EB, W), jnp.float32),
          pltpu.VMEM((EB,), jnp.int32),
          pltpu.VMEM((EB,), jnp.int32),
          pltpu.VMEM((EB, W), jnp.float32),
          pltpu.VMEM((3, SB), jnp.int32),
          pltpu.VMEM((3, SB), jnp.int32),
          pltpu.VMEM((RB, W), jnp.float32),
          pltpu.VMEM((RB, W), jnp.float32),
          pltpu.VMEM((RB, W), jnp.float32),
          pltpu.VMEM((RB, W), jnp.float32),
          pltpu.SemaphoreType.DMA,
          pltpu.SemaphoreType.DMA,
          pltpu.SemaphoreType.DMA,
          pltpu.SemaphoreType.DMA,
          pltpu.SemaphoreType.DMA,
          pltpu.SemaphoreType.DMA,
          pltpu.SemaphoreType.DMA,
          pltpu.SemaphoreType.DMA,
          pltpu.SemaphoreType.DMA,
          pltpu.SemaphoreType.DMA,
          pltpu.SemaphoreType.DMA,
          pltpu.SemaphoreType.DMA,
      ],
      compiler_params=pltpu.CompilerParams(use_tc_tiling_on_sc=False, needs_layout_passes=False),
      name="cheb_spmm",
  )


_spmm_first = _make_spmm(False)
_spmm_cheb = _make_spmm(True)


def _matmul_kernel(x0, x1, x2, x3, x4, wt, bias, out):
  acc = jnp.zeros((out.shape[1], FOUT), jnp.float32)
  for k, xr in enumerate((x0, x1, x2, x3, x4)):
    for h in range(2):
      acc += jnp.dot(xr[h], wt[k, h], preferred_element_type=jnp.float32)
  out[0] = acc + bias[0]


VB = 1000


def _matmul(xs, wt, bias):
  grid = (B, V // VB)
  x_spec = pl.BlockSpec((2, VB, W), lambda b, vb: (b, vb, 0))
  return pl.pallas_call(
      _matmul_kernel,
      grid=grid,
      in_specs=[x_spec] * K + [
          pl.BlockSpec((K, 2, W, FOUT), lambda b, vb: (0, 0, 0, 0)),
          pl.BlockSpec((1, FOUT), lambda b, vb: (0, 0)),
      ],
      out_specs=pl.BlockSpec((1, VB, FOUT), lambda b, vb: (b, vb, 0)),
      out_shape=jax.ShapeDtypeStruct((B, V, FOUT), jnp.float32),
  )(*xs, wt, bias)


def kernel(edge_index, edge_vals, inputs, weight, bias):
  row = edge_index[0]
  col = edge_index[1]
  x0 = inputs.reshape(B, V, 2, W).transpose(0, 2, 1, 3).reshape(NCH * V, W)
  pad = ((0, 0), (0, EPT_P - EPT))
  colp = jnp.pad(col.reshape(NS, EPT), pad).reshape(-1)
  rowp = jnp.pad(row.reshape(NS, EPT), pad).reshape(-1)
  valp = jnp.pad(
      lax.bitcast_convert_type(edge_vals, jnp.int32).reshape(NS, EPT),
      pad).reshape(-1)
  ei3 = jnp.stack([colp, rowp, valp])
  x1 = _spmm_first(x0, ei3, x0)
  x2 = _spmm_cheb(x1, ei3, x0)
  x3 = _spmm_cheb(x2, ei3, x1)
  x4 = _spmm_cheb(x3, ei3, x2)
  wt = weight.transpose(1, 0, 2).reshape(K, 2, W, FOUT)
  xs = [x.reshape(NCH, V, W) for x in (x0, x1, x2, x3, x4)]
  return _matmul(xs, wt, bias.reshape(1, FOUT))


# speedup vs baseline: 1.2348x; 1.2348x over previous
import functools

import jax
import jax.numpy as jnp
from jax import lax
from jax.experimental import pallas as pl
from jax.experimental.pallas import tpu as pltpu
from jax.experimental.pallas import tpu_sc as plsc

B, V, E, FIN, K, FOUT = 4, 10000, 160000, 256, 5, 256

NC, NS, L = 2, 16, 16

W = 128
NCH = (B * FIN) // W
CPC = NCH // NC
EPT = E // NS
EB = 80
NBLK = EPT // EB
RPT = V // NS
RB = 25
NRB = RPT // RB


def _spmm_body(x_hbm, col_hbm, row_hbm, val_hbm, prev_hbm, y_hbm,
               acc,
               col0, row0, val0, gidx0, srow0, rows0,
               col1, row1, val1, gidx1, srow1, rows1,
               col2, row2, val2, gidx2, srow2, rows2,
               wbv0, pbv0, wbv1, pbv1,
               isem0, gsem0, ssem0, isem1, gsem1, ssem1,
               isem2, gsem2, ssem2,
               wsem, lsem0, stsem0, lsem1, stsem1,
               *, has_prev):
  cid = lax.axis_index("c")
  sid = lax.axis_index("s")
  ebase = sid * EPT

  EBUF0 = (col0, row0, val0, gidx0, srow0, rows0, isem0, gsem0, ssem0)
  EBUF1 = (col1, row1, val1, gidx1, srow1, rows1, isem1, gsem1, ssem1)
  EBUF2 = (col2, row2, val2, gidx2, srow2, rows2, isem2, gsem2, ssem2)

  def idx_issue(blk, b):
    off = ebase + blk * EB
    pltpu.async_copy(col_hbm.at[pl.ds(off, EB)], b[0], b[6])
    pltpu.async_copy(row_hbm.at[pl.ds(off, EB)], b[1], b[6])
    pltpu.async_copy(val_hbm.at[pl.ds(off, EB)], b[2], b[6])

  def idx_wait(b):
    pltpu.make_async_copy(col_hbm.at[pl.ds(ebase, EB)], b[0], b[6]).wait()
    pltpu.make_async_copy(row_hbm.at[pl.ds(ebase, EB)], b[1], b[6]).wait()
    pltpu.make_async_copy(val_hbm.at[pl.ds(ebase, EB)], b[2], b[6]).wait()

  def gather_issue(b, cbase):
    for j in range(EB // L):
      sl = pl.ds(j * L, L)
      b[3][sl] = b[0][sl] + cbase
    pltpu.async_copy(x_hbm.at[b[3]], b[5], b[7])

  def gather_wait(b):
    pltpu.make_async_copy(x_hbm.at[b[3]], b[5], b[7]).wait()

  def scatter_issue(b):
    for j in range(EB // L):
      sl = pl.ds(j * L, L)
      b[4][sl] = b[1][sl]
    pltpu.async_copy(b[5], acc.at[b[4]], b[8], add=True)

  def scatter_wait(b):
    pltpu.make_async_copy(b[5], acc.at[b[4]], b[8]).wait()

  def scale(b):
    @pl.loop(0, EB // L)
    def _sg(g):
      v16 = b[2][pl.ds(g * L, L)]
      if has_prev:
        v16 = v16 * 2.0
      for i in range(L):
        val = v16[i]
        e = g * L + i
        for j in range(W // L):
          sl = pl.ds(j * L, L)
          b[5][e, sl] = b[5][e, sl] * val

  def step3(blk, A, Bp, cbase):
    @pl.when(blk + 2 < NBLK)
    def _pf():
      idx_wait(Bp)
      @pl.when(blk >= 1)
      def _dr():
        scatter_wait(Bp)
      gather_issue(Bp, cbase)
    gather_wait(A)
    scale(A)
    scatter_issue(A)
    @pl.when(blk + 3 < NBLK)
    def _nidx():
      idx_issue(blk + 3, A)

  @pl.loop(0, CPC)
  def _chunk_loop(ci):
    chunk = cid * CPC + ci
    cbase = chunk * V

    @pl.loop(0, RB)
    def _zfill(i):
      for j in range(W // L):
        wbv0[i, pl.ds(j * L, L)] = jnp.zeros((L,), jnp.float32)

    @pl.loop(0, NRB)
    def _zissue(wb):
      pltpu.sync_copy(wbv0, acc.at[pl.ds(sid * RPT + wb * RB, RB)])

    plsc.subcore_barrier()

    idx_issue(0, EBUF0)
    idx_issue(1, EBUF1)
    idx_issue(2, EBUF2)
    idx_wait(EBUF0)
    gather_issue(EBUF0, cbase)
    idx_wait(EBUF1)
    gather_issue(EBUF1, cbase)

    @pl.loop(0, NBLK // 3)
    def _trips(i):
      step3(3 * i, EBUF0, EBUF2, cbase)
      step3(3 * i + 1, EBUF1, EBUF0, cbase)
      step3(3 * i + 2, EBUF2, EBUF1, cbase)

    step3(NBLK - 2, EBUF0, EBUF2, cbase)
    step3(NBLK - 1, EBUF1, EBUF0, cbase)
    scatter_wait(EBUF2)
    scatter_wait(EBUF0)
    scatter_wait(EBUF1)

    plsc.subcore_barrier()

    @pl.loop(0, NRB // 2)
    def _wb(i):
      wb = 2 * i
      r0 = sid * RPT + wb * RB
      r1 = r0 + RB
      if has_prev:
        dpa = pltpu.async_copy(prev_hbm.at[pl.ds(cbase + r0, RB)], pbv0, lsem0)
        dpb = pltpu.async_copy(prev_hbm.at[pl.ds(cbase + r1, RB)], pbv1, lsem1)
      pltpu.sync_copy(acc.at[pl.ds(r0, RB)], wbv0)
      if has_prev:
        dpa.wait()

        @pl.loop(0, RB)
        def _sub0(r):
          for j in range(W // L):
            sl = pl.ds(j * L, L)
            wbv0[r, sl] = wbv0[r, sl] - pbv0[r, sl]

      sta = pltpu.async_copy(wbv0, y_hbm.at[pl.ds(cbase + r0, RB)], stsem0)
      pltpu.sync_copy(acc.at[pl.ds(r1, RB)], wbv1)
      if has_prev:
        dpb.wait()

        @pl.loop(0, RB)
        def _sub1(r):
          for j in range(W // L):
            sl = pl.ds(j * L, L)
            wbv1[r, sl] = wbv1[r, sl] - pbv1[r, sl]

      stb = pltpu.async_copy(wbv1, y_hbm.at[pl.ds(cbase + r1, RB)], stsem1)
      sta.wait()
      stb.wait()

    wbt = NRB - 1
    rt = sid * RPT + wbt * RB
    pltpu.sync_copy(acc.at[pl.ds(rt, RB)], wbv0)
    if has_prev:
      pltpu.sync_copy(prev_hbm.at[pl.ds(cbase + rt, RB)], pbv0)

      @pl.loop(0, RB)
      def _subt(r):
        for j in range(W // L):
          sl = pl.ds(j * L, L)
          wbv0[r, sl] = wbv0[r, sl] - pbv0[r, sl]

    pltpu.sync_copy(wbv0, y_hbm.at[pl.ds(cbase + rt, RB)])

    plsc.subcore_barrier()


def _make_spmm(has_prev):
  mesh = plsc.VectorSubcoreMesh(core_axis_name="c", subcore_axis_name="s")
  return pl.kernel(
      functools.partial(_spmm_body, has_prev=has_prev),
      out_type=jax.ShapeDtypeStruct((NCH * V, W), jnp.float32),
      mesh=mesh,
      scratch_types=[
          pltpu.VMEM_SHARED((V, W), jnp.float32),
          pltpu.VMEM((EB,), jnp.int32),
          pltpu.VMEM((EB,), jnp.int32),
          pltpu.VMEM((EB,), jnp.float32),
          pltpu.VMEM((EB,), jnp.int32),
          pltpu.VMEM((EB,), jnp.int32),
          pltpu.VMEM((EB, W), jnp.float32),
          pltpu.VMEM((EB,), jnp.int32),
          pltpu.VMEM((EB,), jnp.int32),
          pltpu.VMEM((EB,), jnp.float32),
          pltpu.VMEM((EB,), jnp.int32),
          pltpu.VMEM((EB,), jnp.int32),
          pltpu.VMEM((EB, W), jnp.float32),
          pltpu.VMEM((EB,), jnp.int32),
          pltpu.VMEM((EB,), jnp.int32),
          pltpu.VMEM((EB,), jnp.float32),
          pltpu.VMEM((EB,), jnp.int32),
          pltpu.VMEM((EB,), jnp.int32),
          pltpu.VMEM((EB, W), jnp.float32),
          pltpu.VMEM((RB, W), jnp.float32),
          pltpu.VMEM((RB, W), jnp.float32),
          pltpu.VMEM((RB, W), jnp.float32),
          pltpu.VMEM((RB, W), jnp.float32),
          pltpu.SemaphoreType.DMA,
          pltpu.SemaphoreType.DMA,
          pltpu.SemaphoreType.DMA,
          pltpu.SemaphoreType.DMA,
          pltpu.SemaphoreType.DMA,
          pltpu.SemaphoreType.DMA,
          pltpu.SemaphoreType.DMA,
          pltpu.SemaphoreType.DMA,
          pltpu.SemaphoreType.DMA,
          pltpu.SemaphoreType.DMA,
          pltpu.SemaphoreType.DMA,
          pltpu.SemaphoreType.DMA,
          pltpu.SemaphoreType.DMA,
          pltpu.SemaphoreType.DMA,
      ],
      compiler_params=pltpu.CompilerParams(use_tc_tiling_on_sc=False),
      name="cheb_spmm",
  )


_spmm_first = _make_spmm(False)
_spmm_cheb = _make_spmm(True)


def _matmul_kernel(x0, x1, x2, x3, x4, wt, bias, out):
  acc = jnp.zeros((out.shape[1], FOUT), jnp.float32)
  for k, xr in enumerate((x0, x1, x2, x3, x4)):
    for h in range(2):
      acc += jnp.dot(xr[h], wt[k, h], preferred_element_type=jnp.float32)
  out[0] = acc + bias[0]


VB = 1000


def _matmul(xs, wt, bias):
  grid = (B, V // VB)
  x_spec = pl.BlockSpec((2, VB, W), lambda b, vb: (b, vb, 0))
  return pl.pallas_call(
      _matmul_kernel,
      grid=grid,
      in_specs=[x_spec] * K + [
          pl.BlockSpec((K, 2, W, FOUT), lambda b, vb: (0, 0, 0, 0)),
          pl.BlockSpec((1, FOUT), lambda b, vb: (0, 0)),
      ],
      out_specs=pl.BlockSpec((1, VB, FOUT), lambda b, vb: (b, vb, 0)),
      out_shape=jax.ShapeDtypeStruct((B, V, FOUT), jnp.float32),
  )(*xs, wt, bias)


def kernel(edge_index, edge_vals, inputs, weight, bias):
  row = edge_index[0]
  col = edge_index[1]
  x0 = inputs.reshape(B, V, 2, W).transpose(0, 2, 1, 3).reshape(NCH * V, W)
  x1 = _spmm_first(x0, col, row, edge_vals, x0)
  x2 = _spmm_cheb(x1, col, row, edge_vals, x0)
  x3 = _spmm_cheb(x2, col, row, edge_vals, x1)
  x4 = _spmm_cheb(x3, col, row, edge_vals, x2)
  wt = weight.transpose(1, 0, 2).reshape(K, 2, W, FOUT)
  xs = [x.reshape(NCH, V, W) for x in (x0, x1, x2, x3, x4)]
  return _matmul(xs, wt, bias.reshape(1, FOUT))
